# baseline (speedup 1.0000x reference)
import functools

import jax
import jax.numpy as jnp
from jax import lax
from jax.experimental import pallas as pl
from jax.experimental.pallas import tpu as pltpu
from jax.experimental.pallas import tpu_sc as plsc

N = 10000
E = 160000
R = 4
SLOPE = 0.01
F0, F1, F2, F3 = 256, 128, 64, 32

NC, NS = 2, 16
NW = NC * NS
EW = E // NW
C = 128
K = (EW + C - 1) // C
PADN = K * C - EW
DUMMY = 240
ACC = N + DUMMY
RPT = ACC // NS
BN = 1000

_MESH = plsc.VectorSubcoreMesh(core_axis_name="c", subcore_axis_name="s",
                               num_cores=NC, num_subcores=NS)


DK = 2 * R * K
DH = 8
DWT = ACC * DH // NS


def _sc_degree(degi, ones_c, zeros_d):
    @functools.partial(
        pl.kernel,
        out_type=jax.ShapeDtypeStruct((NC, ACC * DH), jnp.float32),
        mesh=_MESH,
        compiler_params=pltpu.CompilerParams(skip_device_barrier=True),
        scratch_types=[
            pltpu.VMEM((DK, C), jnp.int32),
            pltpu.VMEM((C,), jnp.float32),
            pltpu.VMEM((DWT,), jnp.float32),
            pltpu.VMEM_SHARED((ACC * DH,), jnp.float32),
        ],
    )
    def deg_kernel(degi_hbm, ones_hbm, zeros_hbm, out_hbm,
                   idx_v, ones_v, zer_v, acc):
        c = lax.axis_index("c")
        s = lax.axis_index("s")
        w = s * NC + c
        pltpu.sync_copy(zeros_hbm, zer_v)
        pltpu.sync_copy(zer_v, acc.at[pl.ds(s * DWT, DWT)])
        pltpu.sync_copy(ones_hbm, ones_v)
        pltpu.sync_copy(degi_hbm.at[w], idx_v)
        plsc.subcore_barrier()

        def body(j, carry):
            pltpu.sync_copy(ones_v, acc.at[idx_v.at[j]], add=True)
            return carry

        lax.fori_loop(0, DK, body, 0)
        plsc.subcore_barrier()
        pltpu.sync_copy(acc.at[pl.ds(s * DWT, DWT)],
                        out_hbm.at[c, pl.ds(s * DWT, DWT)])

    return deg_kernel(degi, ones_c, zeros_d)


def _make_sc_agg(F, groups=1, sc_tiling=False):
    G = groups
    C2 = 128 if F == F1 else 64
    K2 = K * C // C2
    NB = 2 if F == F1 else 4
    @functools.partial(
        pl.kernel,
        out_type=jax.ShapeDtypeStruct((NC, R, ACC, F), jnp.float32),
        mesh=_MESH,
        compiler_params=pltpu.CompilerParams(
            use_tc_tiling_on_sc=False if sc_tiling else None),
        scratch_types=(
            [pltpu.VMEM((K2, C2), jnp.int32),
             pltpu.VMEM((K2, C2), jnp.int32)]
            + [pltpu.VMEM((C2, F), jnp.float32)] * NB
            + [pltpu.VMEM_SHARED((G * ACC, F), jnp.float32)]
            + [pltpu.SemaphoreType.DMA] * (2 * NB)
        ),
    )
    def agg_kernel(h_hbm, srcg_hbm, dsts_hbm, zr_hbm, out_hbm,
                   idxg, idxd, *bufs):
        rows = bufs[:NB]
        acc = bufs[NB]
        gsem = bufs[NB + 1:NB + 1 + NB]
        ssem = bufs[NB + 1 + NB:]
        c = lax.axis_index("c")
        s = lax.axis_index("s")
        w = s * NC + c

        def gath(j, b):
            return pltpu.make_async_copy(h_hbm.at[idxg.at[j]], rows[b],
                                         gsem[b])

        def scat(j, b):
            return pltpu.make_async_copy(rows[b], acc.at[idxd.at[j]],
                                         ssem[b])

        for p in range(R // G):
            pltpu.sync_copy(zr_hbm, rows[0])
            nz = 0
            for g in range(G):
                for k in range(RPT // C2):
                    pltpu.make_async_copy(
                        rows[0],
                        acc.at[pl.ds(g * ACC + s * RPT + k * C2, C2)],
                        gsem[nz % NB]).start()
                    nz += 1
            for q in range(nz):
                pltpu.make_async_copy(
                    rows[0], acc.at[pl.ds(s * RPT, C2)], gsem[q % NB]).wait()
            plsc.subcore_barrier()
            for g in range(G):
                r = p * G + g
                pltpu.sync_copy(srcg_hbm.at[r, w], idxg)
                pltpu.sync_copy(dsts_hbm.at[r, w], idxd)
                gath(0, 0).start()
                gath(1, 1).start()

                if NB == 4:
                    def body(i, carry):
                        for u in range(NB):
                            j = NB * i + u
                            gath(j, u).wait()
                            scat(j, u).start(add=True)
                            bn = (u + 2) % NB

                            @pl.when(j + 2 < K2)
                            def _():
                                @pl.when(j >= 2)
                                def _():
                                    scat(j - 2, bn).wait()
                                gath(j + 2, bn).start()
                        return carry

                    lax.fori_loop(0, K2 // NB, body, 0)
                    for b in range(NB):
                        scat(0, b).wait()
                else:
                    def body(i, carry):
                        for u in range(2):
                            j = 2 * i + u
                            gath(j, u).wait()
                            pltpu.sync_copy(rows[u], acc.at[idxd.at[j]],
                                            add=True)

                            @pl.when(j + 2 < K2)
                            def _():
                                gath(j + 2, u).start()
                        return carry

                    lax.fori_loop(0, K2 // 2, body, 0)
            plsc.subcore_barrier()
            for g in range(G):
                pltpu.sync_copy(acc.at[pl.ds(g * ACC + s * RPT, RPT)],
                                out_hbm.at[c, p * G + g, pl.ds(s * RPT, RPT)])
            if p + 1 < R // G:
                plsc.subcore_barrier()

    return agg_kernel


_sc_agg_1 = _make_sc_agg(F1)
_sc_agg_2 = _make_sc_agg(F2, groups=2, sc_tiling=True)
_sc_agg_3 = _make_sc_agg(F3, groups=4, sc_tiling=True)


def _leaky(v):
    return jnp.maximum(v, SLOPE * v)


def _tc_prep(degp, x, w1):
    def body(degp_ref, x_ref, w_ref, norm_ref, h_ref):
        deg = degp_ref[0] + degp_ref[1]
        nrm = lax.rsqrt(jnp.maximum(deg, 1.0))
        norm_ref[...] = nrm
        for r in range(R):
            xs = x_ref[...] * nrm[:, 2 * r:2 * r + 1]
            h_ref[r] = jnp.dot(xs, w_ref[r],
                               preferred_element_type=jnp.float32)

    return pl.pallas_call(
        body,
        grid=(N // BN,),
        in_specs=[
            pl.BlockSpec((NC, BN, DH), lambda i: (0, i, 0)),
            pl.BlockSpec((BN, F0), lambda i: (i, 0)),
            pl.BlockSpec((R, F0, F1), lambda i: (0, 0, 0)),
        ],
        out_specs=[
            pl.BlockSpec((BN, DH), lambda i: (i, 0)),
            pl.BlockSpec((R, BN, F1), lambda i: (0, i, 0)),
        ],
        out_shape=[
            jax.ShapeDtypeStruct((N, DH), jnp.float32),
            jax.ShapeDtypeStruct((R, N, F1), jnp.float32),
        ],
    )(degp, x, w1)


def _make_tc_mid(F_in, F_out):
    def body(aggp_ref, norm_ref, b_ref, w_ref, h_ref):
        nrm = norm_ref[...]
        tot = None
        for r in range(R):
            y = ((aggp_ref[0, r] + aggp_ref[1, r])
                 * nrm[:, 2 * r + 1:2 * r + 2] + b_ref[r])
            y = _leaky(y)
            tot = y if tot is None else tot + y
        h = _leaky(tot)
        for r in range(R):
            h_ref[r] = jnp.dot(h * nrm[:, 2 * r:2 * r + 1], w_ref[r],
                               preferred_element_type=jnp.float32)

    def call(aggp, norm, b, w):
        return pl.pallas_call(
            body,
            grid=(N // BN,),
            in_specs=[
                pl.BlockSpec((NC, R, BN, F_in), lambda i: (0, 0, i, 0)),
                pl.BlockSpec((BN, DH), lambda i: (i, 0)),
                pl.BlockSpec((R, F_in), lambda i: (0, 0)),
                pl.BlockSpec((R, F_in, F_out), lambda i: (0, 0, 0)),
            ],
            out_specs=pl.BlockSpec((R, BN, F_out), lambda i: (0, i, 0)),
            out_shape=jax.ShapeDtypeStruct((R, N, F_out), jnp.float32),
        )(aggp, norm, b, w)

    return call


_tc_mid_12 = _make_tc_mid(F1, F2)
_tc_mid_23 = _make_tc_mid(F2, F3)


def _tc_final(aggp, norm, b):
    def body(aggp_ref, norm_ref, b_ref, o_ref):
        nrm = norm_ref[...]
        tot = None
        for r in range(R):
            y = ((aggp_ref[0, r] + aggp_ref[1, r])
                 * nrm[:, 2 * r + 1:2 * r + 2] + b_ref[r])
            y = _leaky(y)
            tot = y if tot is None else tot + y
        o_ref[...] = tot

    return pl.pallas_call(
        body,
        grid=(N // BN,),
        in_specs=[
            pl.BlockSpec((NC, R, BN, F3), lambda i: (0, 0, i, 0)),
            pl.BlockSpec((BN, DH), lambda i: (i, 0)),
            pl.BlockSpec((R, F3), lambda i: (0, 0)),
        ],
        out_specs=pl.BlockSpec((BN, F3), lambda i: (i, 0)),
        out_shape=jax.ShapeDtypeStruct((N, F3), jnp.float32),
    )(aggp, norm, b)


def _build_indices(eis):
    ar = jnp.arange(PADN, dtype=jnp.int32)
    gpad = jnp.broadcast_to((ar % N)[None], (NW, PADN))
    spad = jnp.broadcast_to((N + (ar % DUMMY))[None], (NW, PADN))
    srcg, dsts, dega = [], [], []
    for r, ei in enumerate(eis):
        src = ei[0].astype(jnp.int32).reshape(NW, EW)
        dst = ei[1].astype(jnp.int32).reshape(NW, EW)
        srcp = jnp.concatenate([src, spad], 1)
        dstp = jnp.concatenate([dst, spad], 1)
        srcg.append(jnp.concatenate([src + r * N, gpad + r * N], 1))
        dsts.append(dstp)
        dega.append((srcp * DH + 2 * r).reshape(NW, K, C))
        dega.append((dstp * DH + 2 * r + 1).reshape(NW, K, C))
    return (jnp.stack(srcg), jnp.stack(dsts),
            jnp.concatenate(dega, axis=1))


def kernel(x, edge_index_activate, edge_index_repress,
           edge_index_activate_feedback, edge_index_repress_feedback,
           W1, b1, W2, b2, W3, b3):
    eis = [edge_index_activate, edge_index_repress,
           edge_index_activate_feedback, edge_index_repress_feedback]
    srcg, dsts, degi = _build_indices(eis)
    slot = (jnp.arange(R, dtype=jnp.int32) * ACC).reshape(R, 1, 1)
    dsts2 = dsts + slot % (2 * ACC)
    dsts4 = dsts + slot
    degp = _sc_degree(degi, jnp.ones((C,), jnp.float32),
                      jnp.zeros((DWT,), jnp.float32))
    norm, h1 = _tc_prep(degp.reshape(NC, ACC, DH), x, W1)
    agg1 = _sc_agg_1(h1.reshape(R * N, F1),
                     srcg.reshape(R, NW, -1, 128), dsts.reshape(R, NW, -1, 128),
                     jnp.zeros((128, F1), jnp.float32))
    h2 = _tc_mid_12(agg1, norm, b1, W2)
    agg2 = _sc_agg_2(h2.reshape(R * N, F2),
                     srcg.reshape(R, NW, -1, 64), dsts2.reshape(R, NW, -1, 64),
                     jnp.zeros((64, F2), jnp.float32))
    h3 = _tc_mid_23(agg2, norm, b2, W3)
    agg3 = _sc_agg_3(h3.reshape(R * N, F3),
                     srcg.reshape(R, NW, -1, 64), dsts4.reshape(R, NW, -1, 64),
                     jnp.zeros((64, F3), jnp.float32))
    return _tc_final(agg3, norm, b3)


# speedup vs baseline: 1.0741x; 1.0741x over previous
import functools

import jax
import jax.numpy as jnp
from jax import lax
from jax.experimental import pallas as pl
from jax.experimental.pallas import tpu as pltpu
from jax.experimental.pallas import tpu_sc as plsc

N = 10000
E = 160000
R = 4
SLOPE = 0.01
F0, F1, F2, F3 = 256, 128, 64, 32

NC, NS = 2, 16
NW = NC * NS
EW = E // NW
C = 128
K = (EW + C - 1) // C
PADN = K * C - EW
DUMMY = 240
ACC = N + DUMMY
RPT = ACC // NS
BN = 1000

_MESH = plsc.VectorSubcoreMesh(core_axis_name="c", subcore_axis_name="s",
                               num_cores=NC, num_subcores=NS)


DK = 2 * R * K
DH = 8
DWT = ACC * DH // NS


def _sc_degree(degi, ones_c, zeros_d):
    @functools.partial(
        pl.kernel,
        out_type=jax.ShapeDtypeStruct((NC, ACC * DH), jnp.float32),
        mesh=_MESH,
        compiler_params=pltpu.CompilerParams(skip_device_barrier=True),
        scratch_types=[
            pltpu.VMEM((DK, C), jnp.int32),
            pltpu.VMEM((C,), jnp.float32),
            pltpu.VMEM((DWT,), jnp.float32),
            pltpu.VMEM_SHARED((ACC * DH,), jnp.float32),
        ],
    )
    def deg_kernel(degi_hbm, ones_hbm, zeros_hbm, out_hbm,
                   idx_v, ones_v, zer_v, acc):
        c = lax.axis_index("c")
        s = lax.axis_index("s")
        w = s * NC + c
        pltpu.sync_copy(zeros_hbm, zer_v)
        pltpu.sync_copy(zer_v, acc.at[pl.ds(s * DWT, DWT)])
        pltpu.sync_copy(ones_hbm, ones_v)
        pltpu.sync_copy(degi_hbm.at[w], idx_v)
        plsc.subcore_barrier()

        def body(j, carry):
            pltpu.sync_copy(ones_v, acc.at[idx_v.at[j]], add=True)
            return carry

        lax.fori_loop(0, DK, body, 0)
        plsc.subcore_barrier()
        pltpu.sync_copy(acc.at[pl.ds(s * DWT, DWT)],
                        out_hbm.at[c, pl.ds(s * DWT, DWT)])

    return deg_kernel(degi, ones_c, zeros_d)


def _make_sc_agg(F, groups=1, sc_tiling=False):
    G = groups
    @functools.partial(
        pl.kernel,
        out_type=jax.ShapeDtypeStruct((NC, R, ACC, F), jnp.float32),
        mesh=_MESH,
        compiler_params=pltpu.CompilerParams(
            use_tc_tiling_on_sc=False if sc_tiling else None),
        scratch_types=[
            pltpu.VMEM((K, C), jnp.int32),
            pltpu.VMEM((K, C), jnp.int32),
            pltpu.VMEM((C, F), jnp.float32),
            pltpu.VMEM((C, F), jnp.float32),
            pltpu.VMEM_SHARED((G * ACC, F), jnp.float32),
            pltpu.SemaphoreType.DMA,
            pltpu.SemaphoreType.DMA,
        ],
    )
    def agg_kernel(h_hbm, srcg_hbm, dsts_hbm, zr_hbm, out_hbm,
                   idxg, idxd, rows0, rows1, acc, sem0, sem1):
        c = lax.axis_index("c")
        s = lax.axis_index("s")
        w = s * NC + c
        rows = (rows0, rows1)
        sems = (sem0, sem1)
        for p in range(R // G):
            pltpu.sync_copy(zr_hbm, rows0)
            for g in range(G):
                for k in range(RPT // C):
                    pltpu.sync_copy(
                        rows0, acc.at[pl.ds(g * ACC + s * RPT + k * C, C)])
            plsc.subcore_barrier()
            for g in range(G):
                r = p * G + g
                pltpu.sync_copy(srcg_hbm.at[r, w], idxg)
                pltpu.sync_copy(dsts_hbm.at[r, w], idxd)
                pltpu.async_copy(h_hbm.at[idxg.at[0]], rows0, sem0)
                pltpu.async_copy(h_hbm.at[idxg.at[1]], rows1, sem1)

                def body(i, carry):
                    for b in range(2):
                        j = 2 * i + b
                        pltpu.make_async_copy(h_hbm.at[idxg.at[j]], rows[b],
                                              sems[b]).wait()
                        pltpu.sync_copy(rows[b], acc.at[idxd.at[j]], add=True)

                        @pl.when(j + 2 < K)
                        def _():
                            pltpu.async_copy(h_hbm.at[idxg.at[j + 2]],
                                             rows[b], sems[b])
                    return carry

                lax.fori_loop(0, K // 2, body, 0)
            plsc.subcore_barrier()
            for g in range(G):
                pltpu.sync_copy(acc.at[pl.ds(g * ACC + s * RPT, RPT)],
                                out_hbm.at[c, p * G + g, pl.ds(s * RPT, RPT)])
            if p + 1 < R // G:
                plsc.subcore_barrier()

    return agg_kernel


_sc_agg_1 = _make_sc_agg(F1)
_sc_agg_2 = _make_sc_agg(F2, groups=2, sc_tiling=True)
_sc_agg_3 = _make_sc_agg(F3, groups=4, sc_tiling=True)


def _leaky(v):
    return jnp.maximum(v, SLOPE * v)


def _tc_prep(degp, x, w1):
    def body(degp_ref, x_ref, w_ref, norm_ref, h_ref):
        deg = degp_ref[0] + degp_ref[1]
        nrm = lax.rsqrt(jnp.maximum(deg, 1.0))
        norm_ref[...] = nrm
        for r in range(R):
            xs = x_ref[...] * nrm[:, 2 * r:2 * r + 1]
            h_ref[r] = jnp.dot(xs, w_ref[r],
                               preferred_element_type=jnp.float32)

    return pl.pallas_call(
        body,
        grid=(N // BN,),
        in_specs=[
            pl.BlockSpec((NC, BN, DH), lambda i: (0, i, 0)),
            pl.BlockSpec((BN, F0), lambda i: (i, 0)),
            pl.BlockSpec((R, F0, F1), lambda i: (0, 0, 0)),
        ],
        out_specs=[
            pl.BlockSpec((BN, DH), lambda i: (i, 0)),
            pl.BlockSpec((R, BN, F1), lambda i: (0, i, 0)),
        ],
        out_shape=[
            jax.ShapeDtypeStruct((N, DH), jnp.float32),
            jax.ShapeDtypeStruct((R, N, F1), jnp.float32),
        ],
    )(degp, x, w1)


def _make_tc_mid(F_in, F_out):
    def body(aggp_ref, norm_ref, b_ref, w_ref, h_ref):
        nrm = norm_ref[...]
        tot = None
        for r in range(R):
            y = ((aggp_ref[0, r] + aggp_ref[1, r])
                 * nrm[:, 2 * r + 1:2 * r + 2] + b_ref[r])
            y = _leaky(y)
            tot = y if tot is None else tot + y
        h = _leaky(tot)
        for r in range(R):
            h_ref[r] = jnp.dot(h * nrm[:, 2 * r:2 * r + 1], w_ref[r],
                               preferred_element_type=jnp.float32)

    def call(aggp, norm, b, w):
        return pl.pallas_call(
            body,
            grid=(N // BN,),
            in_specs=[
                pl.BlockSpec((NC, R, BN, F_in), lambda i: (0, 0, i, 0)),
                pl.BlockSpec((BN, DH), lambda i: (i, 0)),
                pl.BlockSpec((R, F_in), lambda i: (0, 0)),
                pl.BlockSpec((R, F_in, F_out), lambda i: (0, 0, 0)),
            ],
            out_specs=pl.BlockSpec((R, BN, F_out), lambda i: (0, i, 0)),
            out_shape=jax.ShapeDtypeStruct((R, N, F_out), jnp.float32),
        )(aggp, norm, b, w)

    return call


_tc_mid_12 = _make_tc_mid(F1, F2)
_tc_mid_23 = _make_tc_mid(F2, F3)


def _tc_final(aggp, norm, b):
    def body(aggp_ref, norm_ref, b_ref, o_ref):
        nrm = norm_ref[...]
        tot = None
        for r in range(R):
            y = ((aggp_ref[0, r] + aggp_ref[1, r])
                 * nrm[:, 2 * r + 1:2 * r + 2] + b_ref[r])
            y = _leaky(y)
            tot = y if tot is None else tot + y
        o_ref[...] = tot

    return pl.pallas_call(
        body,
        grid=(N // BN,),
        in_specs=[
            pl.BlockSpec((NC, R, BN, F3), lambda i: (0, 0, i, 0)),
            pl.BlockSpec((BN, DH), lambda i: (i, 0)),
            pl.BlockSpec((R, F3), lambda i: (0, 0)),
        ],
        out_specs=pl.BlockSpec((BN, F3), lambda i: (i, 0)),
        out_shape=jax.ShapeDtypeStruct((N, F3), jnp.float32),
    )(aggp, norm, b)


def _build_indices(eis):
    ar = jnp.arange(PADN, dtype=jnp.int32)
    gpad = jnp.broadcast_to((ar % N)[None], (NW, PADN))
    spad = jnp.broadcast_to((N + (ar % DUMMY))[None], (NW, PADN))
    srcg, dsts, dega = [], [], []
    for r, ei in enumerate(eis):
        src = ei[0].astype(jnp.int32).reshape(NW, EW)
        dst = ei[1].astype(jnp.int32).reshape(NW, EW)
        srcp = jnp.concatenate([src, spad], 1)
        dstp = jnp.concatenate([dst, spad], 1)
        srcg.append(jnp.concatenate([src + r * N, gpad + r * N], 1))
        dsts.append(dstp)
        dega.append((srcp * DH + 2 * r).reshape(NW, K, C))
        dega.append((dstp * DH + 2 * r + 1).reshape(NW, K, C))
    return (jnp.stack(srcg), jnp.stack(dsts),
            jnp.concatenate(dega, axis=1))


def kernel(x, edge_index_activate, edge_index_repress,
           edge_index_activate_feedback, edge_index_repress_feedback,
           W1, b1, W2, b2, W3, b3):
    eis = [edge_index_activate, edge_index_repress,
           edge_index_activate_feedback, edge_index_repress_feedback]
    srcg, dsts, degi = _build_indices(eis)
    slot = (jnp.arange(R, dtype=jnp.int32) * ACC).reshape(R, 1, 1)
    dsts2 = dsts + slot % (2 * ACC)
    dsts4 = dsts + slot
    degp = _sc_degree(degi, jnp.ones((C,), jnp.float32),
                      jnp.zeros((DWT,), jnp.float32))
    norm, h1 = _tc_prep(degp.reshape(NC, ACC, DH), x, W1)
    srcg = srcg.reshape(R, NW, K, C)
    agg1 = _sc_agg_1(h1.reshape(R * N, F1), srcg, dsts.reshape(R, NW, K, C),
                     jnp.zeros((C, F1), jnp.float32))
    h2 = _tc_mid_12(agg1, norm, b1, W2)
    agg2 = _sc_agg_2(h2.reshape(R * N, F2), srcg, dsts2.reshape(R, NW, K, C),
                     jnp.zeros((C, F2), jnp.float32))
    h3 = _tc_mid_23(agg2, norm, b2, W3)
    agg3 = _sc_agg_3(h3.reshape(R * N, F3), srcg, dsts4.reshape(R, NW, K, C),
                     jnp.zeros((C, F3), jnp.float32))
    return _tc_final(agg3, norm, b3)


# speedup vs baseline: 1.0788x; 1.0044x over previous
import functools

import jax
import jax.numpy as jnp
from jax import lax
from jax.experimental import pallas as pl
from jax.experimental.pallas import tpu as pltpu
from jax.experimental.pallas import tpu_sc as plsc

N = 10000
E = 160000
R = 4
SLOPE = 0.01
F0, F1, F2, F3 = 256, 128, 64, 32

NC, NS = 2, 16
NW = NC * NS
EW = E // NW
C = 128
K = (EW + C - 1) // C
PADN = K * C - EW
DUMMY = 240
ACC = N + DUMMY
RPT = ACC // NS
BN = 1000

_MESH = plsc.VectorSubcoreMesh(core_axis_name="c", subcore_axis_name="s",
                               num_cores=NC, num_subcores=NS)


DK = 2 * R * K
DH = 8
DWT = ACC * DH // NS


def _sc_degree(degi, ones_c, zeros_d):
    @functools.partial(
        pl.kernel,
        out_type=jax.ShapeDtypeStruct((NC, ACC * DH), jnp.float32),
        mesh=_MESH,
        compiler_params=pltpu.CompilerParams(skip_device_barrier=True),
        scratch_types=[
            pltpu.VMEM((DK, C), jnp.int32),
            pltpu.VMEM((C,), jnp.float32),
            pltpu.VMEM((DWT,), jnp.float32),
            pltpu.VMEM_SHARED((ACC * DH,), jnp.float32),
        ],
    )
    def deg_kernel(degi_hbm, ones_hbm, zeros_hbm, out_hbm,
                   idx_v, ones_v, zer_v, acc):
        c = lax.axis_index("c")
        s = lax.axis_index("s")
        w = s * NC + c
        pltpu.sync_copy(zeros_hbm, zer_v)
        pltpu.sync_copy(zer_v, acc.at[pl.ds(s * DWT, DWT)])
        pltpu.sync_copy(ones_hbm, ones_v)
        pltpu.sync_copy(degi_hbm.at[w], idx_v)
        plsc.subcore_barrier()

        def body(j, carry):
            pltpu.sync_copy(ones_v, acc.at[idx_v.at[j]], add=True)
            return carry

        lax.fori_loop(0, DK, body, 0)
        plsc.subcore_barrier()
        pltpu.sync_copy(acc.at[pl.ds(s * DWT, DWT)],
                        out_hbm.at[c, pl.ds(s * DWT, DWT)])

    return deg_kernel(degi, ones_c, zeros_d)


def _make_sc_agg(F, groups=1, sc_tiling=False):
    G = groups
    @functools.partial(
        pl.kernel,
        out_type=jax.ShapeDtypeStruct((NC, R, ACC, F), jnp.float32),
        mesh=_MESH,
        compiler_params=pltpu.CompilerParams(
            use_tc_tiling_on_sc=False if sc_tiling else None),
        scratch_types=[
            pltpu.VMEM((K, C), jnp.int32),
            pltpu.VMEM((K, C), jnp.int32),
            pltpu.VMEM((C, F), jnp.float32),
            pltpu.VMEM((C, F), jnp.float32),
            pltpu.VMEM_SHARED((G * ACC, F), jnp.float32),
            pltpu.SemaphoreType.DMA,
            pltpu.SemaphoreType.DMA,
        ],
    )
    def agg_kernel(h_hbm, srcg_hbm, dsts_hbm, zr_hbm, out_hbm,
                   idxg, idxd, rows0, rows1, acc, sem0, sem1):
        c = lax.axis_index("c")
        s = lax.axis_index("s")
        w = s * NC + c
        rows = (rows0, rows1)
        sems = (sem0, sem1)
        pltpu.sync_copy(zr_hbm, rows0)
        for g in range(G):
            for k in range(RPT // C):
                pltpu.sync_copy(
                    rows0, acc.at[pl.ds(g * ACC + s * RPT + k * C, C)])
        plsc.subcore_barrier()
        for p in range(R // G):
            for g in range(G):
                r = p * G + g
                pltpu.sync_copy(srcg_hbm.at[r, w], idxg)
                pltpu.sync_copy(dsts_hbm.at[r, w], idxd)
                pltpu.async_copy(h_hbm.at[idxg.at[0]], rows0, sem0)
                pltpu.async_copy(h_hbm.at[idxg.at[1]], rows1, sem1)

                def body(i, carry):
                    for b in range(2):
                        j = 2 * i + b
                        pltpu.make_async_copy(h_hbm.at[idxg.at[j]], rows[b],
                                              sems[b]).wait()
                        pltpu.sync_copy(rows[b], acc.at[idxd.at[j]], add=True)

                        @pl.when(j + 2 < K)
                        def _():
                            pltpu.async_copy(h_hbm.at[idxg.at[j + 2]],
                                             rows[b], sems[b])
                    return carry

                lax.fori_loop(0, K // 2, body, 0)
            plsc.subcore_barrier()
            for g in range(G):
                pltpu.sync_copy(acc.at[pl.ds(g * ACC + s * RPT, RPT)],
                                out_hbm.at[c, p * G + g, pl.ds(s * RPT, RPT)])
            if p + 1 < R // G:
                pltpu.sync_copy(zr_hbm, rows0)
                for g in range(G):
                    for k in range(RPT // C):
                        pltpu.sync_copy(
                            rows0,
                            acc.at[pl.ds(g * ACC + s * RPT + k * C, C)])
                plsc.subcore_barrier()

    return agg_kernel


_sc_agg_1 = _make_sc_agg(F1)
_sc_agg_2 = _make_sc_agg(F2, groups=2, sc_tiling=True)
_sc_agg_3 = _make_sc_agg(F3, groups=4, sc_tiling=True)


def _leaky(v):
    return jnp.maximum(v, SLOPE * v)


def _tc_prep(degp, x, w1):
    def body(degp_ref, x_ref, w_ref, norm_ref, h_ref):
        deg = degp_ref[0] + degp_ref[1]
        nrm = lax.rsqrt(jnp.maximum(deg, 1.0))
        norm_ref[...] = nrm
        for r in range(R):
            xs = x_ref[...] * nrm[:, 2 * r:2 * r + 1]
            h_ref[r] = jnp.dot(xs, w_ref[r],
                               preferred_element_type=jnp.float32)

    return pl.pallas_call(
        body,
        grid=(N // BN,),
        in_specs=[
            pl.BlockSpec((NC, BN, DH), lambda i: (0, i, 0)),
            pl.BlockSpec((BN, F0), lambda i: (i, 0)),
            pl.BlockSpec((R, F0, F1), lambda i: (0, 0, 0)),
        ],
        out_specs=[
            pl.BlockSpec((BN, DH), lambda i: (i, 0)),
            pl.BlockSpec((R, BN, F1), lambda i: (0, i, 0)),
        ],
        out_shape=[
            jax.ShapeDtypeStruct((N, DH), jnp.float32),
            jax.ShapeDtypeStruct((R, N, F1), jnp.float32),
        ],
    )(degp, x, w1)


def _make_tc_mid(F_in, F_out):
    def body(aggp_ref, norm_ref, b_ref, w_ref, h_ref):
        nrm = norm_ref[...]
        tot = None
        for r in range(R):
            y = ((aggp_ref[0, r] + aggp_ref[1, r])
                 * nrm[:, 2 * r + 1:2 * r + 2] + b_ref[r])
            y = _leaky(y)
            tot = y if tot is None else tot + y
        h = _leaky(tot)
        for r in range(R):
            h_ref[r] = jnp.dot(h * nrm[:, 2 * r:2 * r + 1], w_ref[r],
                               preferred_element_type=jnp.float32)

    def call(aggp, norm, b, w):
        return pl.pallas_call(
            body,
            grid=(N // BN,),
            in_specs=[
                pl.BlockSpec((NC, R, BN, F_in), lambda i: (0, 0, i, 0)),
                pl.BlockSpec((BN, DH), lambda i: (i, 0)),
                pl.BlockSpec((R, F_in), lambda i: (0, 0)),
                pl.BlockSpec((R, F_in, F_out), lambda i: (0, 0, 0)),
            ],
            out_specs=pl.BlockSpec((R, BN, F_out), lambda i: (0, i, 0)),
            out_shape=jax.ShapeDtypeStruct((R, N, F_out), jnp.float32),
        )(aggp, norm, b, w)

    return call


_tc_mid_12 = _make_tc_mid(F1, F2)
_tc_mid_23 = _make_tc_mid(F2, F3)


def _tc_final(aggp, norm, b):
    def body(aggp_ref, norm_ref, b_ref, o_ref):
        nrm = norm_ref[...]
        tot = None
        for r in range(R):
            y = ((aggp_ref[0, r] + aggp_ref[1, r])
                 * nrm[:, 2 * r + 1:2 * r + 2] + b_ref[r])
            y = _leaky(y)
            tot = y if tot is None else tot + y
        o_ref[...] = tot

    return pl.pallas_call(
        body,
        grid=(N // BN,),
        in_specs=[
            pl.BlockSpec((NC, R, BN, F3), lambda i: (0, 0, i, 0)),
            pl.BlockSpec((BN, DH), lambda i: (i, 0)),
            pl.BlockSpec((R, F3), lambda i: (0, 0)),
        ],
        out_specs=pl.BlockSpec((BN, F3), lambda i: (i, 0)),
        out_shape=jax.ShapeDtypeStruct((N, F3), jnp.float32),
    )(aggp, norm, b)


def _build_indices(eis):
    ar = jnp.arange(PADN, dtype=jnp.int32)
    gpad = jnp.broadcast_to((ar % N)[None], (NW, PADN))
    spad = jnp.broadcast_to((N + (ar % DUMMY))[None], (NW, PADN))
    srcg, dsts, dega = [], [], []
    for r, ei in enumerate(eis):
        src = ei[0].astype(jnp.int32).reshape(NW, EW)
        dst = ei[1].astype(jnp.int32).reshape(NW, EW)
        srcp = jnp.concatenate([src, spad], 1)
        dstp = jnp.concatenate([dst, spad], 1)
        srcg.append(jnp.concatenate([src + r * N, gpad + r * N], 1))
        dsts.append(dstp)
        dega.append((srcp * DH + 2 * r).reshape(NW, K, C))
        dega.append((dstp * DH + 2 * r + 1).reshape(NW, K, C))
    return (jnp.stack(srcg), jnp.stack(dsts),
            jnp.concatenate(dega, axis=1))


def kernel(x, edge_index_activate, edge_index_repress,
           edge_index_activate_feedback, edge_index_repress_feedback,
           W1, b1, W2, b2, W3, b3):
    eis = [edge_index_activate, edge_index_repress,
           edge_index_activate_feedback, edge_index_repress_feedback]
    srcg, dsts, degi = _build_indices(eis)
    slot = (jnp.arange(R, dtype=jnp.int32) * ACC).reshape(R, 1, 1)
    dsts2 = dsts + slot % (2 * ACC)
    dsts4 = dsts + slot
    degp = _sc_degree(degi, jnp.ones((C,), jnp.float32),
                      jnp.zeros((DWT,), jnp.float32))
    norm, h1 = _tc_prep(degp.reshape(NC, ACC, DH), x, W1)
    srcg = srcg.reshape(R, NW, K, C)
    agg1 = _sc_agg_1(h1.reshape(R * N, F1), srcg, dsts.reshape(R, NW, K, C),
                     jnp.zeros((C, F1), jnp.float32))
    h2 = _tc_mid_12(agg1, norm, b1, W2)
    agg2 = _sc_agg_2(h2.reshape(R * N, F2), srcg, dsts2.reshape(R, NW, K, C),
                     jnp.zeros((C, F2), jnp.float32))
    h3 = _tc_mid_23(agg2, norm, b2, W3)
    agg3 = _sc_agg_3(h3.reshape(R * N, F3), srcg, dsts4.reshape(R, NW, K, C),
                     jnp.zeros((C, F3), jnp.float32))
    return _tc_final(agg3, norm, b3)


# speedup vs baseline: 1.0804x; 1.0015x over previous
import functools

import jax
import jax.numpy as jnp
from jax import lax
from jax.experimental import pallas as pl
from jax.experimental.pallas import tpu as pltpu
from jax.experimental.pallas import tpu_sc as plsc

N = 10000
E = 160000
R = 4
SLOPE = 0.01
F0, F1, F2, F3 = 256, 128, 64, 32

NC, NS = 2, 16
NW = NC * NS
EW = E // NW
C = 128
K = (EW + C - 1) // C
PADN = K * C - EW
DUMMY = 240
ACC = N + DUMMY
RPT = ACC // NS
BN = 2000

_MESH = plsc.VectorSubcoreMesh(core_axis_name="c", subcore_axis_name="s",
                               num_cores=NC, num_subcores=NS)


DK = 2 * R * K
DH = 8
DWT = ACC * DH // NS


def _sc_degree(degi, ones_c, zeros_d):
    @functools.partial(
        pl.kernel,
        out_type=jax.ShapeDtypeStruct((NC, ACC * DH), jnp.float32),
        mesh=_MESH,
        compiler_params=pltpu.CompilerParams(skip_device_barrier=True),
        scratch_types=[
            pltpu.VMEM((DK, C), jnp.int32),
            pltpu.VMEM((C,), jnp.float32),
            pltpu.VMEM((DWT,), jnp.float32),
            pltpu.VMEM_SHARED((ACC * DH,), jnp.float32),
        ],
    )
    def deg_kernel(degi_hbm, ones_hbm, zeros_hbm, out_hbm,
                   idx_v, ones_v, zer_v, acc):
        c = lax.axis_index("c")
        s = lax.axis_index("s")
        w = s * NC + c
        pltpu.sync_copy(zeros_hbm, zer_v)
        pltpu.sync_copy(zer_v, acc.at[pl.ds(s * DWT, DWT)])
        pltpu.sync_copy(ones_hbm, ones_v)
        pltpu.sync_copy(degi_hbm.at[w], idx_v)
        plsc.subcore_barrier()

        def body(j, carry):
            pltpu.sync_copy(ones_v, acc.at[idx_v.at[j]], add=True)
            return carry

        lax.fori_loop(0, DK, body, 0)
        plsc.subcore_barrier()
        pltpu.sync_copy(acc.at[pl.ds(s * DWT, DWT)],
                        out_hbm.at[c, pl.ds(s * DWT, DWT)])

    return deg_kernel(degi, ones_c, zeros_d)


def _make_sc_agg(F, groups=1, sc_tiling=False):
    G = groups
    @functools.partial(
        pl.kernel,
        out_type=jax.ShapeDtypeStruct((NC, R, ACC, F), jnp.float32),
        mesh=_MESH,
        compiler_params=pltpu.CompilerParams(
            use_tc_tiling_on_sc=False if sc_tiling else None),
        scratch_types=[
            pltpu.VMEM((K, C), jnp.int32),
            pltpu.VMEM((K, C), jnp.int32),
            pltpu.VMEM((C, F), jnp.float32),
            pltpu.VMEM((C, F), jnp.float32),
            pltpu.VMEM_SHARED((G * ACC, F), jnp.float32),
            pltpu.SemaphoreType.DMA,
            pltpu.SemaphoreType.DMA,
        ],
    )
    def agg_kernel(h_hbm, srcg_hbm, dsts_hbm, zr_hbm, out_hbm,
                   idxg, idxd, rows0, rows1, acc, sem0, sem1):
        c = lax.axis_index("c")
        s = lax.axis_index("s")
        w = s * NC + c
        rows = (rows0, rows1)
        sems = (sem0, sem1)
        pltpu.sync_copy(zr_hbm, rows0)
        for g in range(G):
            for k in range(RPT // C):
                pltpu.sync_copy(
                    rows0, acc.at[pl.ds(g * ACC + s * RPT + k * C, C)])
        plsc.subcore_barrier()
        for p in range(R // G):
            for g in range(G):
                r = p * G + g
                pltpu.sync_copy(srcg_hbm.at[r, w], idxg)
                pltpu.sync_copy(dsts_hbm.at[r, w], idxd)
                pltpu.async_copy(h_hbm.at[idxg.at[0]], rows0, sem0)
                pltpu.async_copy(h_hbm.at[idxg.at[1]], rows1, sem1)

                def body(i, carry):
                    for b in range(2):
                        j = 2 * i + b
                        pltpu.make_async_copy(h_hbm.at[idxg.at[j]], rows[b],
                                              sems[b]).wait()
                        pltpu.sync_copy(rows[b], acc.at[idxd.at[j]], add=True)

                        @pl.when(j + 2 < K)
                        def _():
                            pltpu.async_copy(h_hbm.at[idxg.at[j + 2]],
                                             rows[b], sems[b])
                    return carry

                lax.fori_loop(0, K // 2, body, 0)
            plsc.subcore_barrier()
            for g in range(G):
                pltpu.sync_copy(acc.at[pl.ds(g * ACC + s * RPT, RPT)],
                                out_hbm.at[c, p * G + g, pl.ds(s * RPT, RPT)])
            if p + 1 < R // G:
                pltpu.sync_copy(zr_hbm, rows0)
                for g in range(G):
                    for k in range(RPT // C):
                        pltpu.sync_copy(
                            rows0,
                            acc.at[pl.ds(g * ACC + s * RPT + k * C, C)])
                plsc.subcore_barrier()

    return agg_kernel


_sc_agg_1 = _make_sc_agg(F1)
_sc_agg_2 = _make_sc_agg(F2, groups=2, sc_tiling=True)
_sc_agg_3 = _make_sc_agg(F3, groups=4, sc_tiling=True)


def _leaky(v):
    return jnp.maximum(v, SLOPE * v)


def _tc_prep(degp, x, w1):
    def body(degp_ref, x_ref, w_ref, norm_ref, h_ref):
        deg = degp_ref[0] + degp_ref[1]
        nrm = lax.rsqrt(jnp.maximum(deg, 1.0))
        norm_ref[...] = nrm
        for r in range(R):
            xs = x_ref[...] * nrm[:, 2 * r:2 * r + 1]
            h_ref[r] = jnp.dot(xs, w_ref[r],
                               preferred_element_type=jnp.float32)

    return pl.pallas_call(
        body,
        grid=(N // BN,),
        in_specs=[
            pl.BlockSpec((NC, BN, DH), lambda i: (0, i, 0)),
            pl.BlockSpec((BN, F0), lambda i: (i, 0)),
            pl.BlockSpec((R, F0, F1), lambda i: (0, 0, 0)),
        ],
        out_specs=[
            pl.BlockSpec((BN, DH), lambda i: (i, 0)),
            pl.BlockSpec((R, BN, F1), lambda i: (0, i, 0)),
        ],
        out_shape=[
            jax.ShapeDtypeStruct((N, DH), jnp.float32),
            jax.ShapeDtypeStruct((R, N, F1), jnp.float32),
        ],
    )(degp, x, w1)


def _make_tc_mid(F_in, F_out):
    def body(aggp_ref, norm_ref, b_ref, w_ref, h_ref):
        nrm = norm_ref[...]
        tot = None
        for r in range(R):
            y = ((aggp_ref[0, r] + aggp_ref[1, r])
                 * nrm[:, 2 * r + 1:2 * r + 2] + b_ref[r])
            y = _leaky(y)
            tot = y if tot is None else tot + y
        h = _leaky(tot)
        for r in range(R):
            h_ref[r] = jnp.dot(h * nrm[:, 2 * r:2 * r + 1], w_ref[r],
                               preferred_element_type=jnp.float32)

    def call(aggp, norm, b, w):
        return pl.pallas_call(
            body,
            grid=(N // BN,),
            in_specs=[
                pl.BlockSpec((NC, R, BN, F_in), lambda i: (0, 0, i, 0)),
                pl.BlockSpec((BN, DH), lambda i: (i, 0)),
                pl.BlockSpec((R, F_in), lambda i: (0, 0)),
                pl.BlockSpec((R, F_in, F_out), lambda i: (0, 0, 0)),
            ],
            out_specs=pl.BlockSpec((R, BN, F_out), lambda i: (0, i, 0)),
            out_shape=jax.ShapeDtypeStruct((R, N, F_out), jnp.float32),
        )(aggp, norm, b, w)

    return call


_tc_mid_12 = _make_tc_mid(F1, F2)
_tc_mid_23 = _make_tc_mid(F2, F3)


def _tc_final(aggp, norm, b):
    def body(aggp_ref, norm_ref, b_ref, o_ref):
        nrm = norm_ref[...]
        tot = None
        for r in range(R):
            y = ((aggp_ref[0, r] + aggp_ref[1, r])
                 * nrm[:, 2 * r + 1:2 * r + 2] + b_ref[r])
            y = _leaky(y)
            tot = y if tot is None else tot + y
        o_ref[...] = tot

    return pl.pallas_call(
        body,
        grid=(N // BN,),
        in_specs=[
            pl.BlockSpec((NC, R, BN, F3), lambda i: (0, 0, i, 0)),
            pl.BlockSpec((BN, DH), lambda i: (i, 0)),
            pl.BlockSpec((R, F3), lambda i: (0, 0)),
        ],
        out_specs=pl.BlockSpec((BN, F3), lambda i: (i, 0)),
        out_shape=jax.ShapeDtypeStruct((N, F3), jnp.float32),
    )(aggp, norm, b)


def _build_indices(eis):
    ar = jnp.arange(PADN, dtype=jnp.int32)
    gpad = jnp.broadcast_to((ar % N)[None], (NW, PADN))
    spad = jnp.broadcast_to((N + (ar % DUMMY))[None], (NW, PADN))
    srcg, dsts, dega = [], [], []
    for r, ei in enumerate(eis):
        src = ei[0].astype(jnp.int32).reshape(NW, EW)
        dst = ei[1].astype(jnp.int32).reshape(NW, EW)
        srcp = jnp.concatenate([src, spad], 1)
        dstp = jnp.concatenate([dst, spad], 1)
        srcg.append(jnp.concatenate([src + r * N, gpad + r * N], 1))
        dsts.append(dstp)
        dega.append((srcp * DH + 2 * r).reshape(NW, K, C))
        dega.append((dstp * DH + 2 * r + 1).reshape(NW, K, C))
    return (jnp.stack(srcg), jnp.stack(dsts),
            jnp.concatenate(dega, axis=1))


def kernel(x, edge_index_activate, edge_index_repress,
           edge_index_activate_feedback, edge_index_repress_feedback,
           W1, b1, W2, b2, W3, b3):
    eis = [edge_index_activate, edge_index_repress,
           edge_index_activate_feedback, edge_index_repress_feedback]
    srcg, dsts, degi = _build_indices(eis)
    slot = (jnp.arange(R, dtype=jnp.int32) * ACC).reshape(R, 1, 1)
    dsts2 = dsts + slot % (2 * ACC)
    dsts4 = dsts + slot
    degp = _sc_degree(degi, jnp.ones((C,), jnp.float32),
                      jnp.zeros((DWT,), jnp.float32))
    norm, h1 = _tc_prep(degp.reshape(NC, ACC, DH), x, W1)
    srcg = srcg.reshape(R, NW, K, C)
    agg1 = _sc_agg_1(h1.reshape(R * N, F1), srcg, dsts.reshape(R, NW, K, C),
                     jnp.zeros((C, F1), jnp.float32))
    h2 = _tc_mid_12(agg1, norm, b1, W2)
    agg2 = _sc_agg_2(h2.reshape(R * N, F2), srcg, dsts2.reshape(R, NW, K, C),
                     jnp.zeros((C, F2), jnp.float32))
    h3 = _tc_mid_23(agg2, norm, b2, W3)
    agg3 = _sc_agg_3(h3.reshape(R * N, F3), srcg, dsts4.reshape(R, NW, K, C),
                     jnp.zeros((C, F3), jnp.float32))
    return _tc_final(agg3, norm, b3)
